# baseline (speedup 1.0000x reference)
import functools

import jax
import jax.numpy as jnp
from jax import lax
from jax.experimental import pallas as pl
from jax.experimental.pallas import tpu as pltpu
from jax.experimental.pallas import tpu_sc as plsc

VOCAB = 1000000
EMBED = 64
PAD_IDX = 0

NC, NS, L = 2, 16, 16
NW = NC * NS
B_TOTAL = 4096 * 50
B_PER_W = B_TOTAL // NW
G = 128
ROWS_PER_W = B_PER_W // G
CHUNK_G = 5
CHUNK = CHUNK_G * G
NCHUNK = B_PER_W // CHUNK


def _emb_kernel(idx_hbm, table_hbm, out_hbm, idx_v, rows_v, gsem):
    wid = lax.axis_index("s") * NC + lax.axis_index("c")
    wbase = wid * B_PER_W

    pltpu.sync_copy(idx_hbm.at[wid], idx_v)

    def _min_body(q, mv):
        for k in range(G // L):
            mv = jnp.minimum(mv, idx_v[q, pl.ds(k * L, L)])
        return mv

    mv = lax.fori_loop(0, ROWS_PER_W, _min_body,
                       jnp.full((L,), VOCAB, jnp.int32))
    m = mv[0]
    for l in range(1, L):
        m = jnp.minimum(m, mv[l])
    has_pad = m == 0

    zeros = jnp.zeros((L,), jnp.float32)

    for c in range(NCHUNK):
        copies = [
            pltpu.async_copy(
                table_hbm.at[idx_v.at[c * CHUNK_G + j]],
                rows_v.at[pl.ds(j * G, G)],
                gsem,
            )
            for j in range(CHUNK_G)
        ]
        for cp in copies:
            cp.wait()

        @pl.when(has_pad)
        def _fixup(c=c):
            def body(g, carry):
                gi0 = c * CHUNK + g * L
                v = idx_v[gi0 // G, pl.ds(gi0 % G, L)]

                for r in range(L):
                    @pl.when(v[r] == PAD_IDX)
                    def _zero(r=r):
                        for cc in range(EMBED // L):
                            rows_v[g * L + r, pl.ds(cc * L, L)] = zeros

                return carry

            lax.fori_loop(0, CHUNK // L, body, 0)

        pltpu.sync_copy(rows_v, out_hbm.at[pl.ds(wbase + c * CHUNK, CHUNK)])


@jax.jit
def kernel(src_indices, src_table):
    idx = src_indices.reshape(NW, ROWS_PER_W, G).astype(jnp.int32)
    mesh = plsc.VectorSubcoreMesh(core_axis_name="c", subcore_axis_name="s")
    out = pl.kernel(
        _emb_kernel,
        out_type=jax.ShapeDtypeStruct((B_TOTAL, EMBED), jnp.float32),
        mesh=mesh,
        scratch_types=[
            pltpu.VMEM((ROWS_PER_W, G), jnp.int32),
            pltpu.VMEM((CHUNK, EMBED), jnp.float32),
            pltpu.SemaphoreType.DMA,
        ],
        compiler_params=pltpu.CompilerParams(use_tc_tiling_on_sc=False),
    )(idx, src_table)
    return out.reshape(4096, 50, EMBED)
